# baseline (speedup 1.0000x reference)
import functools

import jax
import jax.numpy as jnp
from jax import lax
from jax.experimental import pallas as pl
from jax.experimental.pallas import tpu as pltpu
from jax.experimental.pallas import tpu_sc as plsc

N = 10000
E = 320000
D = 128
NEG = 0.2

NC, NS, L = 2, 16, 16
NW = NC * NS
EPW = E // NW
K = 80
NCH = EPW // K
NACC = 10240
NPT = NACC // NS
NLAST = N - (NS - 1) * NPT
NDEN = 10112
DLAST = NDEN - (NS - 1) * NPT


def _prep_body(feat_ref, wt_ref, b_ref, al_ref, ar_ref,
               z_ref, el_ref, er_ref, c_ref):
    z = jnp.dot(feat_ref[...], wt_ref[...],
                preferred_element_type=jnp.float32) + b_ref[...]
    z_ref[...] = z
    el = jnp.dot(z, al_ref[...], preferred_element_type=jnp.float32)
    er = jnp.dot(z, ar_ref[...], preferred_element_type=jnp.float32)
    el_ref[...] = el
    er_ref[...] = er
    c = jnp.maximum(jnp.max(el) + jnp.max(er), 0.0)
    c_ref[...] = jnp.full((1, L), 0.0) + c


def _edge_body(row_hbm, col_hbm, elf_hbm, erf_hbm, cvec_hbm, z_hbm,
               num_hbm, den_hbm,
               row_v, col_v, rowsc_v, w_v, el_v, er_v, c_v, rows_v, stag_v,
               accum, dacc, semi, semz, sems):
    cid = lax.axis_index("c")
    sid = lax.axis_index("s")
    wid = sid * NC + cid

    pltpu.sync_copy(elf_hbm, el_v)
    pltpu.sync_copy(erf_hbm, er_v)
    pltpu.sync_copy(cvec_hbm, c_v)

    cvec = c_v[...]

    zv = cvec * 0.0

    def zrow_body(e, carry):
        for q in range(D // L):
            stag_v[e, pl.ds(q * L, L)] = zv
        return carry

    lax.fori_loop(0, K, zrow_body, 0)
    for t in range(K // L):
        w_v[0, pl.ds(t * L, L)] = zv
    for q in range(NPT // K):
        pltpu.sync_copy(stag_v, accum.at[pl.ds(sid * NPT + q * K, K)])
        pltpu.sync_copy(w_v.at[0], dacc.at[pl.ds(sid * NPT + q * K, K)])

    def logit(s):
        for t in range(K // L):
            ridx = row_v[s, pl.ds(t * L, L)]
            cidx = col_v[s, pl.ds(t * L, L)]
            x = plsc.load_gather(el_v, [ridx]) + plsc.load_gather(er_v, [cidx])
            a = jnp.maximum(x, x * NEG) - cvec
            w_v[s, pl.ds(t * L, L)] = jnp.exp(a)

    def scale(s):
        for t in range(K // L):
            rowsc_v[s, pl.ds(t * L, L)] = row_v[s, pl.ds(t * L, L)]

        def scale_body(i, c2):
            for u in range(4):
                e = i * 4 + u
                wsp = plsc.load_gather(
                    w_v,
                    [jnp.full((L,), s, jnp.int32), jnp.full((L,), e, jnp.int32)])
                for q in range(D // (2 * L)):
                    v32 = plsc.bitcast(rows_v[s, e, pl.ds(q * L, L)],
                                       jnp.bfloat16)
                    lo, hi = plsc.unpack(
                        v32, format=plsc.PackFormat.INTERLEAVED)
                    stag_v[e, pl.ds(q * 2 * L, L)] = lo * wsp
                    stag_v[e, pl.ds(q * 2 * L + L, L)] = hi * wsp
            return c2

        lax.fori_loop(0, K // 4, scale_body, 0)

    def scatter(s):
        pltpu.async_copy(stag_v, accum.at[rowsc_v.at[s]], sems, add=True)
        pltpu.async_copy(w_v.at[s], dacc.at[rowsc_v.at[s]], sems, add=True)

    def drain_scatter(s):
        pltpu.make_async_copy(num_hbm.at[0, pl.ds(0, K)], stag_v, sems).wait()
        pltpu.make_async_copy(elf_hbm.at[pl.ds(0, K)], w_v.at[s], sems).wait()

    def drain_idx(s):
        pltpu.make_async_copy(row_hbm.at[0], row_v.at[s], semi).wait()
        pltpu.make_async_copy(col_hbm.at[0], col_v.at[s], semi).wait()

    def drain_z(s):
        pltpu.make_async_copy(z_hbm.at[pl.ds(0, K)], rows_v.at[s], semz).wait()

    plsc.subcore_barrier()

    base = wid * NCH
    pltpu.sync_copy(row_hbm.at[base], row_v.at[0])
    pltpu.sync_copy(col_hbm.at[base], col_v.at[0])
    logit(0)
    pltpu.async_copy(z_hbm.at[col_v.at[0]], rows_v.at[0], semz)
    pltpu.async_copy(row_hbm.at[base + 1], row_v.at[1], semi)
    pltpu.async_copy(col_hbm.at[base + 1], col_v.at[1], semi)

    def pipe_body(i, carry):
        for b in range(2):
            j = 2 * i + b

            @pl.when(j > 0)
            def _():
                drain_scatter(1 - b)

            drain_idx(1 - b)
            pltpu.async_copy(z_hbm.at[col_v.at[1 - b]], rows_v.at[1 - b], semz)
            drain_z(b)
            scale(b)
            scatter(b)
            nxt = base + jnp.minimum(j + 2, NCH - 1)
            pltpu.async_copy(row_hbm.at[nxt], row_v.at[b], semi)
            pltpu.async_copy(col_hbm.at[nxt], col_v.at[b], semi)
            logit(1 - b)
        return carry

    lax.fori_loop(0, (NCH - 1) // 2, pipe_body, 0)

    drain_z(0)
    scale(0)
    drain_scatter(1)
    scatter(0)
    drain_idx(1)
    drain_scatter(0)

    plsc.subcore_barrier()

    @pl.when(sid < NS - 1)
    def _():
        pltpu.sync_copy(accum.at[pl.ds(sid * NPT, NPT)],
                        num_hbm.at[cid, pl.ds(sid * NPT, NPT)])
        pltpu.sync_copy(dacc.at[pl.ds(sid * NPT, NPT)],
                        den_hbm.at[cid, pl.ds(sid * NPT, NPT)])

    @pl.when(sid == NS - 1)
    def _():
        pltpu.sync_copy(accum.at[pl.ds((NS - 1) * NPT, NLAST)],
                        num_hbm.at[cid, pl.ds((NS - 1) * NPT, NLAST)])
        pltpu.sync_copy(dacc.at[pl.ds((NS - 1) * NPT, DLAST)],
                        den_hbm.at[cid, pl.ds((NS - 1) * NPT, DLAST)])


_edge_kernel = functools.partial(
    pl.kernel,
    out_type=[
        jax.ShapeDtypeStruct((NC, N, D), jnp.float32),
        jax.ShapeDtypeStruct((NC, NDEN), jnp.float32),
    ],
    mesh=plsc.VectorSubcoreMesh(core_axis_name="c", subcore_axis_name="s"),
    compiler_params=pltpu.CompilerParams(needs_layout_passes=False,
                                         use_tc_tiling_on_sc=False),
    scratch_types=[
        pltpu.VMEM((2, K), jnp.int32),
        pltpu.VMEM((2, K), jnp.int32),
        pltpu.VMEM((2, K), jnp.int32),
        pltpu.VMEM((2, K), jnp.float32),
        pltpu.VMEM((N,), jnp.float32),
        pltpu.VMEM((N,), jnp.float32),
        pltpu.VMEM((L,), jnp.float32),
        pltpu.VMEM((2, K, D // 2), jnp.int32),

        pltpu.VMEM((K, D), jnp.float32),
        pltpu.VMEM_SHARED((NACC, D), jnp.float32),
        pltpu.VMEM_SHARED((NACC,), jnp.float32),
        pltpu.SemaphoreType.DMA,
        pltpu.SemaphoreType.DMA,
        pltpu.SemaphoreType.DMA,
    ],
)(_edge_body)


def _combine_body(p_ref, d_ref, o_ref):
    num = p_ref[0] + p_ref[1]
    den = d_ref[0] + d_ref[1]
    o_ref[...] = jnp.where(den > 0.0, num / den, 0.0)


@jax.jit
def kernel(feat, row, col, W, b, a_l, a_r):
    z, el, er, c = pl.pallas_call(
        _prep_body,
        out_shape=[
            jax.ShapeDtypeStruct((N, D), jnp.float32),
            jax.ShapeDtypeStruct((N, 1), jnp.float32),
            jax.ShapeDtypeStruct((N, 1), jnp.float32),
            jax.ShapeDtypeStruct((1, L), jnp.float32),
        ],
    )(feat, W.T, b.reshape(1, D), a_l.reshape(D, 1), a_r.reshape(D, 1))

    row2 = row.reshape(NW * NCH, K)
    col2 = col.reshape(NW * NCH, K)
    zb = jnp.swapaxes(z.reshape(N, D // 32, 2, L), 2, 3)
    zb = zb.reshape(N, D).astype(jnp.bfloat16)
    z32 = lax.bitcast_convert_type(zb.reshape(N, D // 2, 2), jnp.int32)

    num, den = _edge_kernel(row2, col2, el.reshape(N), er.reshape(N),
                            c.reshape(L), z32)

    out = pl.pallas_call(
        _combine_body,
        out_shape=jax.ShapeDtypeStruct((N, D), jnp.float32),
    )(num, den[:, :N].reshape(NC, N, 1))
    return out


# speedup vs baseline: 1.0024x; 1.0024x over previous
import functools

import jax
import jax.numpy as jnp
from jax import lax
from jax.experimental import pallas as pl
from jax.experimental.pallas import tpu as pltpu
from jax.experimental.pallas import tpu_sc as plsc

N = 10000
E = 320000
D = 128
DH = D // 2
NEG = 0.2

NC, NS, L = 2, 16, 16
NW = NC * NS
EPW = E // NW
K = 80
NCH = EPW // K
NACC = 10240
NPT = NACC // NS
NLAST = N - (NS - 1) * NPT
NDEN = 10112
DLAST = NDEN - (NS - 1) * NPT


def _prep_body(feat_ref, wt_ref, b_ref, al_ref, ar_ref,
               z_ref, el_ref, er_ref, c_ref):
    z = jnp.dot(feat_ref[...], wt_ref[...],
                preferred_element_type=jnp.float32) + b_ref[...]
    z_ref[...] = z
    el = jnp.dot(z, al_ref[...], preferred_element_type=jnp.float32)
    er = jnp.dot(z, ar_ref[...], preferred_element_type=jnp.float32)
    el_ref[...] = el
    er_ref[...] = er
    c = jnp.maximum(jnp.max(el) + jnp.max(er), 0.0)
    c_ref[...] = jnp.full((1, L), 0.0) + c


def _edge_body(row_hbm, col_hbm, elf_hbm, erf_hbm, cvec_hbm, z_hbm,
               num_hbm, den_hbm,
               row_v, col_v, rowsc_v, w_v, elg_v, erg_v, c_v, rows_v, stag_v,
               accum, dacc, semi, semz, semg, sems):
    cid = lax.axis_index("c")
    sid = lax.axis_index("s")
    wid = sid * NC + cid

    pltpu.sync_copy(cvec_hbm, c_v)
    cvec = c_v[...]

    zv = cvec * 0.0

    def zrow_body(e, carry):
        for q in range(D // L):
            stag_v[0, e, pl.ds(q * L, L)] = zv
        return carry

    lax.fori_loop(0, K, zrow_body, 0)
    for t in range(K // L):
        w_v[0, pl.ds(t * L, L)] = zv
    for q in range(NPT // K):
        pltpu.sync_copy(stag_v.at[0], accum.at[pl.ds(sid * NPT + q * K, K)])
        pltpu.sync_copy(w_v.at[0], dacc.at[pl.ds(sid * NPT + q * K, K)])

    def logit(s):
        for t in range(K // L):
            x = elg_v[s, pl.ds(t * L, L)] + erg_v[s, pl.ds(t * L, L)]
            a = jnp.maximum(x, x * NEG) - cvec
            w_v[s, pl.ds(t * L, L)] = jnp.exp(a)

    def scale(s):
        for t in range(K // L):
            rowsc_v[s, pl.ds(t * L, L)] = row_v[s, pl.ds(t * L, L)]

        def scale_body(i, c2):
            for u in range(4):
                e = i * 4 + u
                wsp = plsc.load_gather(
                    w_v,
                    [jnp.full((L,), s, jnp.int32), jnp.full((L,), e, jnp.int32)])
                for q in range(DH // L):
                    v32 = plsc.bitcast(rows_v[s, e, pl.ds(q * L, L)],
                                       jnp.bfloat16)
                    lo, hi = plsc.unpack(
                        v32, format=plsc.PackFormat.INTERLEAVED)
                    stag_v[s, e, pl.ds(q * 2 * L, L)] = lo * wsp
                    stag_v[s, e, pl.ds(q * 2 * L + L, L)] = hi * wsp
            return c2

        lax.fori_loop(0, K // 4, scale_body, 0)

    def scatter(s):
        pltpu.async_copy(stag_v.at[s], accum.at[rowsc_v.at[s]], sems, add=True)
        pltpu.async_copy(w_v.at[s], dacc.at[rowsc_v.at[s]], sems, add=True)

    def gathers(s, ch):
        pltpu.async_copy(z_hbm.at[col_v.at[s]], rows_v.at[s], semz)
        pltpu.async_copy(elf_hbm.at[row_v.at[s]], elg_v.at[s], semg)
        pltpu.async_copy(erf_hbm.at[col_v.at[s]], erg_v.at[s], semg)
        del ch

    def drain_scatter(s):
        pltpu.make_async_copy(num_hbm.at[0, pl.ds(0, K)], stag_v.at[s],
                              sems).wait()
        pltpu.make_async_copy(elf_hbm.at[pl.ds(0, K)], w_v.at[s], sems).wait()

    def drain_eg(s):
        pltpu.make_async_copy(elf_hbm.at[pl.ds(0, K)], elg_v.at[s],
                              semg).wait()
        pltpu.make_async_copy(elf_hbm.at[pl.ds(0, K)], erg_v.at[s],
                              semg).wait()

    def drain_idx(s):
        pltpu.make_async_copy(row_hbm.at[0], row_v.at[s], semi).wait()
        pltpu.make_async_copy(col_hbm.at[0], col_v.at[s], semi).wait()

    def drain_z(s):
        pltpu.make_async_copy(z_hbm.at[pl.ds(0, K)], rows_v.at[s], semz).wait()

    plsc.subcore_barrier()

    base = wid * NCH
    pltpu.sync_copy(row_hbm.at[base], row_v.at[0])
    pltpu.sync_copy(col_hbm.at[base], col_v.at[0])
    gathers(0, base)
    pltpu.async_copy(row_hbm.at[base + 1], row_v.at[1], semi)
    pltpu.async_copy(col_hbm.at[base + 1], col_v.at[1], semi)

    def pipe_body(i, carry):
        for b in range(2):
            j = 2 * i + b

            @pl.when(j > 0)
            def _():
                drain_scatter(1 - b)

            drain_idx(1 - b)
            gathers(1 - b, 0)
            drain_eg(b)
            logit(b)
            drain_z(b)
            scale(b)
            scatter(b)
            nxt = base + jnp.minimum(j + 2, NCH - 1)
            pltpu.async_copy(row_hbm.at[nxt], row_v.at[b], semi)
            pltpu.async_copy(col_hbm.at[nxt], col_v.at[b], semi)
        return carry

    lax.fori_loop(0, (NCH - 1) // 2, pipe_body, 0)

    drain_scatter(1)
    drain_eg(0)
    logit(0)
    drain_z(0)
    scale(0)
    scatter(0)
    drain_idx(1)
    drain_scatter(0)

    plsc.subcore_barrier()

    @pl.when(sid < NS - 1)
    def _():
        pltpu.sync_copy(accum.at[pl.ds(sid * NPT, NPT)],
                        num_hbm.at[cid, pl.ds(sid * NPT, NPT)])
        pltpu.sync_copy(dacc.at[pl.ds(sid * NPT, NPT)],
                        den_hbm.at[cid, pl.ds(sid * NPT, NPT)])

    @pl.when(sid == NS - 1)
    def _():
        pltpu.sync_copy(accum.at[pl.ds((NS - 1) * NPT, NLAST)],
                        num_hbm.at[cid, pl.ds((NS - 1) * NPT, NLAST)])
        pltpu.sync_copy(dacc.at[pl.ds((NS - 1) * NPT, DLAST)],
                        den_hbm.at[cid, pl.ds((NS - 1) * NPT, DLAST)])


_edge_kernel = functools.partial(
    pl.kernel,
    out_type=[
        jax.ShapeDtypeStruct((NC, N, D), jnp.float32),
        jax.ShapeDtypeStruct((NC, NDEN), jnp.float32),
    ],
    mesh=plsc.VectorSubcoreMesh(core_axis_name="c", subcore_axis_name="s"),
    compiler_params=pltpu.CompilerParams(needs_layout_passes=False,
                                         use_tc_tiling_on_sc=False),
    scratch_types=[
        pltpu.VMEM((2, K), jnp.int32),
        pltpu.VMEM((2, K), jnp.int32),
        pltpu.VMEM((2, K), jnp.int32),
        pltpu.VMEM((2, K), jnp.float32),
        pltpu.VMEM((2, K), jnp.float32),
        pltpu.VMEM((2, K), jnp.float32),
        pltpu.VMEM((L,), jnp.float32),
        pltpu.VMEM((2, K, DH), jnp.int32),
        pltpu.VMEM((2, K, D), jnp.float32),
        pltpu.VMEM_SHARED((NACC, D), jnp.float32),
        pltpu.VMEM_SHARED((NACC,), jnp.float32),
        pltpu.SemaphoreType.DMA,
        pltpu.SemaphoreType.DMA,
        pltpu.SemaphoreType.DMA,
        pltpu.SemaphoreType.DMA,
    ],
)(_edge_body)


def _combine_body(p_ref, d_ref, o_ref):
    num = p_ref[0] + p_ref[1]
    den = d_ref[0] + d_ref[1]
    o_ref[...] = jnp.where(den > 0.0, num / den, 0.0)


@jax.jit
def kernel(feat, row, col, W, b, a_l, a_r):
    z, el, er, c = pl.pallas_call(
        _prep_body,
        out_shape=[
            jax.ShapeDtypeStruct((N, D), jnp.float32),
            jax.ShapeDtypeStruct((N, 1), jnp.float32),
            jax.ShapeDtypeStruct((N, 1), jnp.float32),
            jax.ShapeDtypeStruct((1, L), jnp.float32),
        ],
    )(feat, W.T, b.reshape(1, D), a_l.reshape(D, 1), a_r.reshape(D, 1))

    row2 = row.reshape(NW * NCH, K)
    col2 = col.reshape(NW * NCH, K)
    zb = jnp.swapaxes(z.reshape(N, D // 32, 2, L), 2, 3)
    zb = zb.reshape(N, D).astype(jnp.bfloat16)
    z32 = lax.bitcast_convert_type(zb.reshape(N, DH, 2), jnp.int32)

    num, den = _edge_kernel(row2, col2, el.reshape(N), er.reshape(N),
                            c.reshape(L), z32)

    out = pl.pallas_call(
        _combine_body,
        out_shape=jax.ShapeDtypeStruct((N, D), jnp.float32),
    )(num, den[:, :N].reshape(NC, N, 1))
    return out


# speedup vs baseline: 1.6732x; 1.6692x over previous
import functools

import jax
import jax.numpy as jnp
from jax import lax
from jax.experimental import pallas as pl
from jax.experimental.pallas import tpu as pltpu
from jax.experimental.pallas import tpu_sc as plsc

N = 10000
E = 320000
D = 128
NEG = 0.2

NC, NS, L = 2, 16, 16
NW = NC * NS
EPW = E // NW
K = 80
NCH = EPW // K
NACC = 10240
NPT = NACC // NS
NLAST = N - (NS - 1) * NPT
NDEN = 10112
DLAST = NDEN - (NS - 1) * NPT


def _prep_body(feat_ref, wt_ref, b_ref, al_ref, ar_ref,
               z_ref, el_ref, er_ref, c_ref):
    z = jnp.dot(feat_ref[...], wt_ref[...],
                preferred_element_type=jnp.float32) + b_ref[...]
    z_ref[...] = z
    el = jnp.dot(z, al_ref[...], preferred_element_type=jnp.float32)
    er = jnp.dot(z, ar_ref[...], preferred_element_type=jnp.float32)
    el_ref[...] = el
    er_ref[...] = er
    c = jnp.maximum(jnp.max(el) + jnp.max(er), 0.0)
    c_ref[...] = jnp.full((1, L), 0.0) + c


def _edge_body(row_hbm, col_hbm, elf_hbm, erf_hbm, cvec_hbm, z_hbm,
               num_hbm, den_hbm,
               row_v, col_v, rowsc_v, w_v, el_v, er_v, c_v, rows_v,
               accum, dacc, semi, semz, sems):
    cid = lax.axis_index("c")
    sid = lax.axis_index("s")
    wid = sid * NC + cid

    pltpu.sync_copy(elf_hbm, el_v)
    pltpu.sync_copy(erf_hbm, er_v)
    pltpu.sync_copy(cvec_hbm, c_v)

    cvec = c_v[...]

    zv = cvec * 0.0

    def zrow_body(e, carry):
        for q in range(D // L):
            rows_v[1, e, pl.ds(q * L, L)] = zv
        return carry

    lax.fori_loop(0, K, zrow_body, 0)
    for t in range(K // L):
        w_v[0, pl.ds(t * L, L)] = zv
    for q in range(NPT // K):
        pltpu.sync_copy(rows_v.at[1], accum.at[pl.ds(sid * NPT + q * K, K)])
        pltpu.sync_copy(w_v.at[0], dacc.at[pl.ds(sid * NPT + q * K, K)])

    def logit(s):
        for t in range(K // L):
            ridx = row_v[s, pl.ds(t * L, L)]
            cidx = col_v[s, pl.ds(t * L, L)]
            x = plsc.load_gather(el_v, [ridx]) + plsc.load_gather(er_v, [cidx])
            a = jnp.maximum(x, x * NEG) - cvec
            w_v[s, pl.ds(t * L, L)] = jnp.exp(a)

    def scale(s):
        for t in range(K // L):
            rowsc_v[s, pl.ds(t * L, L)] = row_v[s, pl.ds(t * L, L)]

        def scale_body(i, c2):
            for u in range(4):
                e = i * 4 + u
                wsp = plsc.load_gather(
                    w_v,
                    [jnp.full((L,), s, jnp.int32), jnp.full((L,), e, jnp.int32)])
                for q in range(D // L):
                    rows_v[s, e, pl.ds(q * L, L)] = (
                        rows_v[s, e, pl.ds(q * L, L)] * wsp)
            return c2

        lax.fori_loop(0, K // 4, scale_body, 0)

    def scatter(s):
        pltpu.async_copy(rows_v.at[s], accum.at[rowsc_v.at[s]], sems, add=True)
        pltpu.async_copy(w_v.at[s], dacc.at[rowsc_v.at[s]], sems, add=True)

    def drain_scatter(s):
        pltpu.make_async_copy(z_hbm.at[pl.ds(0, K)], rows_v.at[s], sems).wait()
        pltpu.make_async_copy(elf_hbm.at[pl.ds(0, K)], w_v.at[s], sems).wait()

    def drain_idx(s):
        pltpu.make_async_copy(row_hbm.at[0], row_v.at[s], semi).wait()
        pltpu.make_async_copy(col_hbm.at[0], col_v.at[s], semi).wait()

    def drain_z(s):
        pltpu.make_async_copy(z_hbm.at[pl.ds(0, K)], rows_v.at[s], semz).wait()

    plsc.subcore_barrier()

    base = wid * NCH
    pltpu.sync_copy(row_hbm.at[base], row_v.at[0])
    pltpu.sync_copy(col_hbm.at[base], col_v.at[0])
    logit(0)
    pltpu.async_copy(z_hbm.at[col_v.at[0]], rows_v.at[0], semz)
    pltpu.async_copy(row_hbm.at[base + 1], row_v.at[1], semi)
    pltpu.async_copy(col_hbm.at[base + 1], col_v.at[1], semi)

    def pipe_body(i, carry):
        for b in range(2):
            j = 2 * i + b

            @pl.when(j > 0)
            def _():
                drain_scatter(1 - b)

            drain_idx(1 - b)
            pltpu.async_copy(z_hbm.at[col_v.at[1 - b]], rows_v.at[1 - b], semz)
            drain_z(b)
            scale(b)
            scatter(b)
            nxt = base + jnp.minimum(j + 2, NCH - 1)
            pltpu.async_copy(row_hbm.at[nxt], row_v.at[b], semi)
            pltpu.async_copy(col_hbm.at[nxt], col_v.at[b], semi)
            logit(1 - b)
        return carry

    lax.fori_loop(0, (NCH - 1) // 2, pipe_body, 0)

    drain_z(0)
    scale(0)
    drain_scatter(1)
    scatter(0)
    drain_idx(1)
    drain_scatter(0)

    plsc.subcore_barrier()

    @pl.when(sid < NS - 1)
    def _():
        pltpu.sync_copy(accum.at[pl.ds(sid * NPT, NPT)],
                        num_hbm.at[cid, pl.ds(sid * NPT, NPT)])
        pltpu.sync_copy(dacc.at[pl.ds(sid * NPT, NPT)],
                        den_hbm.at[cid, pl.ds(sid * NPT, NPT)])

    @pl.when(sid == NS - 1)
    def _():
        pltpu.sync_copy(accum.at[pl.ds((NS - 1) * NPT, NLAST)],
                        num_hbm.at[cid, pl.ds((NS - 1) * NPT, NLAST)])
        pltpu.sync_copy(dacc.at[pl.ds((NS - 1) * NPT, DLAST)],
                        den_hbm.at[cid, pl.ds((NS - 1) * NPT, DLAST)])


_edge_kernel = functools.partial(
    pl.kernel,
    out_type=[
        jax.ShapeDtypeStruct((NC, N, D), jnp.float32),
        jax.ShapeDtypeStruct((NC, NDEN), jnp.float32),
    ],
    mesh=plsc.VectorSubcoreMesh(core_axis_name="c", subcore_axis_name="s"),
    compiler_params=pltpu.CompilerParams(needs_layout_passes=False,
                                         use_tc_tiling_on_sc=False),
    scratch_types=[
        pltpu.VMEM((2, K), jnp.int32),
        pltpu.VMEM((2, K), jnp.int32),
        pltpu.VMEM((2, K), jnp.int32),
        pltpu.VMEM((2, K), jnp.float32),
        pltpu.VMEM((N,), jnp.float32),
        pltpu.VMEM((N,), jnp.float32),
        pltpu.VMEM((L,), jnp.float32),
        pltpu.VMEM((2, K, D), jnp.float32),
        pltpu.VMEM_SHARED((NACC, D), jnp.float32),
        pltpu.VMEM_SHARED((NACC,), jnp.float32),
        pltpu.SemaphoreType.DMA,
        pltpu.SemaphoreType.DMA,
        pltpu.SemaphoreType.DMA,
    ],
)(_edge_body)


def _combine_body(p_ref, d_ref, o_ref):
    num = p_ref[0] + p_ref[1]
    den = d_ref[0] + d_ref[1]
    o_ref[...] = jnp.where(den > 0.0, num / den, 0.0)


@jax.jit
def kernel(feat, row, col, W, b, a_l, a_r):
    z, el, er, c = pl.pallas_call(
        _prep_body,
        out_shape=[
            jax.ShapeDtypeStruct((N, D), jnp.float32),
            jax.ShapeDtypeStruct((N, 1), jnp.float32),
            jax.ShapeDtypeStruct((N, 1), jnp.float32),
            jax.ShapeDtypeStruct((1, L), jnp.float32),
        ],
    )(feat, W.T, b.reshape(1, D), a_l.reshape(D, 1), a_r.reshape(D, 1))

    row2 = row.reshape(NW * NCH, K)
    col2 = col.reshape(NW * NCH, K)

    num, den = _edge_kernel(row2, col2, el.reshape(N), er.reshape(N),
                            c.reshape(L), z)

    out = pl.pallas_call(
        _combine_body,
        out_shape=jax.ShapeDtypeStruct((N, D), jnp.float32),
    )(num, den[:, :N].reshape(NC, N, 1))
    return out
